# baseline (speedup 1.0000x reference)
import jax
import jax.numpy as jnp
from jax.experimental import pallas as pl
from jax.experimental.pallas import tpu as pltpu


def _body(slot_ref, q_ref, k_ref, v_ref, slope_ref, kv_ref, out_ref):
    b = pl.program_id(0)
    q = q_ref[0]
    k = k_ref[0]
    v = v_ref[0]
    kv = kv_ref[0]
    decay = jnp.exp(-slope_ref[...])
    qk = jnp.sum(q * k, axis=-1, keepdims=True)
    ctx = jnp.sum(q[:, :, None] * kv, axis=1)
    out = decay * ctx + qk * v
    valid = slot_ref[b] >= 0
    out_ref[0] = jnp.where(valid, out, 0.0)


def kernel(q, k, v, kv_cache, slope_rate, slot_idx):
    B, H, _, D = q.shape
    q3 = q.reshape(B, H, D)
    k3 = k.reshape(B, H, D)
    v3 = v.reshape(B, H, D)
    slope2 = slope_rate.reshape(H, 1)

    grid_spec = pltpu.PrefetchScalarGridSpec(
        num_scalar_prefetch=1,
        grid=(B,),
        in_specs=[
            pl.BlockSpec((1, H, D), lambda b, s: (b, 0, 0)),
            pl.BlockSpec((1, H, D), lambda b, s: (b, 0, 0)),
            pl.BlockSpec((1, H, D), lambda b, s: (b, 0, 0)),
            pl.BlockSpec((H, 1), lambda b, s: (0, 0)),
            pl.BlockSpec((1, H, D, D),
                         lambda b, s: (jnp.maximum(s[b], 0), 0, 0, 0)),
        ],
        out_specs=pl.BlockSpec((1, H, D), lambda b, s: (b, 0, 0)),
    )
    out = pl.pallas_call(
        _body,
        grid_spec=grid_spec,
        out_shape=jax.ShapeDtypeStruct((B, H, D), jnp.float32),
    )(slot_idx, q3, k3, v3, slope2, kv_cache)
    return out[:, :, None, :]
